# baseline (speedup 1.0000x reference)
import functools

import jax
import jax.numpy as jnp
from jax import lax
from jax.experimental import pallas as pl
from jax.experimental.pallas import tpu as pltpu
from jax.experimental.pallas import tpu_sc as plsc

N_FILTER = 256
KEEP = 48 * N_FILTER
ROWS = 1024
NW = 32
ROWS_PER_W = ROWS // NW
CHUNK = 4
NCHUNK = ROWS_PER_W // CHUNK


def _sc_copy(in_hbm, out_hbm, buf0, buf1, s_in0, s_in1, s_out0, s_out1):
    wid = lax.axis_index("s") * 2 + lax.axis_index("c")
    base = wid * ROWS_PER_W
    bufs = (buf0, buf1)
    s_in = (s_in0, s_in1)
    s_out = (s_out0, s_out1)
    for i in range(NCHUNK):
        b = i % 2
        r = base + i * CHUNK
        if i >= 2:
            pltpu.make_async_copy(
                bufs[b], out_hbm.at[pl.ds(r - 2 * CHUNK, CHUNK), :], s_out[b]
            ).wait()
        pltpu.async_copy(
            in_hbm.at[pl.ds(r, CHUNK), pl.ds(0, KEEP)], bufs[b], s_in[b]
        ).wait()
        pltpu.async_copy(bufs[b], out_hbm.at[pl.ds(r, CHUNK), :], s_out[b])
    for i in range(NCHUNK - 2, NCHUNK):
        b = i % 2
        r = base + i * CHUNK
        pltpu.make_async_copy(
            bufs[b], out_hbm.at[pl.ds(r, CHUNK), :], s_out[b]
        ).wait()


def kernel(inputs):
    mesh = plsc.VectorSubcoreMesh(core_axis_name="c", subcore_axis_name="s")
    k = functools.partial(
        pl.kernel,
        mesh=mesh,
        out_type=jax.ShapeDtypeStruct((ROWS, KEEP), inputs.dtype),
        scratch_types=[
            pltpu.VMEM((CHUNK, KEEP), jnp.float32),
            pltpu.VMEM((CHUNK, KEEP), jnp.float32),
            pltpu.SemaphoreType.DMA,
            pltpu.SemaphoreType.DMA,
            pltpu.SemaphoreType.DMA,
            pltpu.SemaphoreType.DMA,
        ],
    )(_sc_copy)
    return k(inputs)


# speedup vs baseline: 1.6565x; 1.6565x over previous
import jax
import jax.numpy as jnp
from jax.experimental import pallas as pl

N_FILTER = 256
KEEP = 48 * N_FILTER
BLOCK_ROWS = 64


def _copy_kernel(in_ref, out_ref):
    out_ref[...] = in_ref[...]


def kernel(inputs):
    rows = inputs.shape[0]
    grid = (rows // BLOCK_ROWS,)
    return pl.pallas_call(
        _copy_kernel,
        grid=grid,
        in_specs=[
            pl.BlockSpec((BLOCK_ROWS, KEEP), lambda i: (i, 0)),
        ],
        out_specs=pl.BlockSpec((BLOCK_ROWS, KEEP), lambda i: (i, 0)),
        out_shape=jax.ShapeDtypeStruct((rows, KEEP), inputs.dtype),
    )(inputs)


# speedup vs baseline: 1.6889x; 1.0195x over previous
import jax
import jax.numpy as jnp
from jax.experimental import pallas as pl
from jax.experimental.pallas import tpu as pltpu

N_FILTER = 256
KEEP = 48 * N_FILTER


def _copy_kernel(in_ref, out_ref, buf, s_in, s_out):
    pltpu.async_copy(
        in_ref.at[:, pl.ds(0, KEEP)], buf, s_in
    ).wait()
    pltpu.async_copy(buf, out_ref, s_out).wait()


def kernel(inputs):
    rows = inputs.shape[0]
    return pl.pallas_call(
        _copy_kernel,
        in_specs=[pl.BlockSpec(memory_space=pltpu.MemorySpace.HBM)],
        out_specs=pl.BlockSpec(memory_space=pltpu.MemorySpace.HBM),
        out_shape=jax.ShapeDtypeStruct((rows, KEEP), inputs.dtype),
        scratch_shapes=[
            pltpu.VMEM((rows, KEEP), jnp.float32),
            pltpu.SemaphoreType.DMA,
            pltpu.SemaphoreType.DMA,
        ],
    )(inputs)


# speedup vs baseline: 1.7070x; 1.0107x over previous
import jax
import jax.numpy as jnp
from jax.experimental import pallas as pl

N_FILTER = 256
KEEP = 48 * N_FILTER
BLOCK_ROWS = 256


def _copy_kernel(in_ref, out_ref):
    out_ref[...] = in_ref[...]


def kernel(inputs):
    rows = inputs.shape[0]
    grid = (rows // BLOCK_ROWS,)
    return pl.pallas_call(
        _copy_kernel,
        grid=grid,
        in_specs=[
            pl.BlockSpec((BLOCK_ROWS, KEEP), lambda i: (i, 0)),
        ],
        out_specs=pl.BlockSpec((BLOCK_ROWS, KEEP), lambda i: (i, 0)),
        out_shape=jax.ShapeDtypeStruct((rows, KEEP), inputs.dtype),
    )(inputs)
